# baseline (speedup 1.0000x reference)
import functools

import jax
import jax.numpy as jnp
import numpy as np
from jax import lax
from jax.experimental import pallas as pl
from jax.experimental.pallas import tpu as pltpu
from jax.experimental.pallas import tpu_sc as plsc

_NC = 2
_NS = 16
_NW = _NC * _NS
_L = 16

_B = 16384
_D = 16
_BPW = _B // _NW
_CH = _BPW // 128
_GROUPS = _BPW // _L


def _sc_body(iu_hbm, iv_hbm, table_hbm, sig_hbm, d2_hbm, s_hbm,
             idxu_v, idxv_v, us_v, vs_v, sig_v, sigb_v, d2_v, s_v, sem):
    wid = lax.axis_index("s") * _NC + lax.axis_index("c")
    base = wid * _BPW
    pltpu.sync_copy(iu_hbm.at[wid], idxu_v)
    pltpu.sync_copy(iv_hbm.at[wid], idxv_v)
    pltpu.sync_copy(sig_hbm, sig_v)
    copies = []
    for c in range(_CH):
        copies.append(pltpu.async_copy(
            table_hbm.at[idxu_v.at[c]], us_v.at[pl.ds(c * 128, 128)], sem))
        copies.append(pltpu.async_copy(
            table_hbm.at[idxv_v.at[c]], vs_v.at[pl.ds(c * 128, 128)], sem))
    ones = jnp.ones((_L,), jnp.float32)
    sig_vec = sig_v[...]
    for d in range(_D):
        sigb_v[pl.ds(d * _L, _L)] = ones / (sig_vec[d] * ones)
    for cp in copies:
        cp.wait()

    iota = lax.iota(jnp.int32, _L)

    def group(g, carry):
        rows = g * _L + iota
        d2 = jnp.zeros((_L,), jnp.float32)
        ss = jnp.zeros((_L,), jnp.float32)
        for d in range(_D):
            cols = jnp.full((_L,), d, dtype=jnp.int32)
            tu = plsc.load_gather(us_v, [rows, cols])
            tv = plsc.load_gather(vs_v, [rows, cols])
            diff = tu - tv
            d2 = d2 + diff * diff
            ss = ss + (tu * tu + tv * tv) * sigb_v[pl.ds(d * _L, _L)]
        off = pl.multiple_of(g * _L, _L)
        d2_v[pl.ds(off, _L)] = d2
        s_v[pl.ds(off, _L)] = ss
        return carry

    lax.fori_loop(0, _GROUPS, group, 0)
    pltpu.sync_copy(d2_v, d2_hbm.at[pl.ds(base, _BPW)])
    pltpu.sync_copy(s_v, s_hbm.at[pl.ds(base, _BPW)])


@functools.cache
def _make_sc_call():
    @functools.partial(
        pl.kernel,
        mesh=plsc.VectorSubcoreMesh(core_axis_name="c", subcore_axis_name="s"),
        compiler_params=pltpu.CompilerParams(
            needs_layout_passes=False, use_tc_tiling_on_sc=False),
        out_type=[
            jax.ShapeDtypeStruct((_B,), jnp.float32),
            jax.ShapeDtypeStruct((_B,), jnp.float32),
        ],
        scratch_types=[
            pltpu.VMEM((_CH, 128), jnp.int32),
            pltpu.VMEM((_CH, 128), jnp.int32),
            pltpu.VMEM((_BPW, _D), jnp.float32),
            pltpu.VMEM((_BPW, _D), jnp.float32),
            pltpu.VMEM((_D,), jnp.float32),
            pltpu.VMEM((_D * _L,), jnp.float32),
            pltpu.VMEM((_BPW,), jnp.float32),
            pltpu.VMEM((_BPW,), jnp.float32),
            pltpu.SemaphoreType.DMA,
        ],
    )
    def _sc_call(iu_hbm, iv_hbm, table_hbm, sig_hbm, d2_hbm, s_hbm, *scratch):
        _sc_body(iu_hbm, iv_hbm, table_hbm, sig_hbm, d2_hbm, s_hbm, *scratch)

    return _sc_call


def _tc_body(bg_ref, sig_ref, d2_ref, s_ref, lab_ref, out_ref):
    beta = bg_ref[0]
    gamma = bg_ref[1]
    const2 = _D * jnp.log(jnp.float32(2.0 * np.pi)) + jnp.sum(jnp.log(sig_ref[...]))
    dist = jnp.sqrt(d2_ref[...])
    x = beta * dist - gamma
    sp = jnp.maximum(x, 0.0) + jnp.log1p(jnp.exp(-jnp.abs(x)))
    sn = sp - x
    latent = (const2 + 0.5 * s_ref[...]) * jnp.float32(1.0 / (1000000 - 1))
    out_ref[...] = jnp.where(lab_ref[...] == 1, sp, sn) + latent


def _tc_call(bg, sig, d2, ss, lab):
    return pl.pallas_call(
        _tc_body,
        out_shape=jax.ShapeDtypeStruct((128, 128), jnp.float32),
        in_specs=[
            pl.BlockSpec(memory_space=pltpu.SMEM),
            pl.BlockSpec(memory_space=pltpu.VMEM),
            pl.BlockSpec(memory_space=pltpu.VMEM),
            pl.BlockSpec(memory_space=pltpu.VMEM),
            pl.BlockSpec(memory_space=pltpu.VMEM),
        ],
    )(bg, sig, d2, ss, lab)


def kernel(pairs, labels, table, sigma, beta, gamma):
    iu = pairs[:, 0].reshape(_NW, _CH, 128)
    iv = pairs[:, 1].reshape(_NW, _CH, 128)
    d2, ss = _make_sc_call()(iu, iv, table, sigma)
    bg = jnp.stack([beta, gamma]).astype(jnp.float32)
    loss = _tc_call(bg, sigma.reshape(1, _D), d2.reshape(128, 128),
                    ss.reshape(128, 128), labels.reshape(128, 128))
    return loss.reshape(_B)
